# baseline (speedup 1.0000x reference)
import jax
import jax.numpy as jnp
from jax.experimental import pallas as pl
from jax.experimental.pallas import tpu as pltpu

B, T, DIM = 1, 2048, 768
HEADS, DHEAD = 12, 64
HDIM = HEADS * DHEAD
CBS, SBS = 4, 4
NSEL, NMEM = 4, 1
WINDOW = 32
SCALE = 0.12
CDIM = CBS * DHEAD
HID = CDIM * 4
NBLK = T // CBS

TQ = 256
TK = 256
QT = T // TQ
CKP = 640
NEG = -1e30
WPAD = 32
WCOLS = 320
TW = WPAD + T + (WCOLS - TK - WPAD)

f32 = jnp.float32
bf16 = jnp.bfloat16


def _dot(a, b, precision=jax.lax.Precision.DEFAULT):
    return jax.lax.dot_general(a, b, (((1,), (0,)), ((), ())),
                               preferred_element_type=f32,
                               precision=precision)


def _dot_nt(a, b, precision=jax.lax.Precision.DEFAULT):
    return jax.lax.dot_general(a, b, (((1,), (1,)), ((), ())),
                               preferred_element_type=f32,
                               precision=precision)


def _pairswap(x):
    even = jax.lax.broadcasted_iota(jnp.int32, x.shape, 1) % 2 == 0
    return jnp.where(even, jnp.roll(x, -1, axis=1), jnp.roll(x, 1, axis=1))


def _qkv_kernel(x_ref, w3_ref, cos_ref, sin_ref,
                q_ref, k_ref, kb_ref, vb_ref):
    hp = pl.program_id(1)
    xb = x_ref[...]
    z = _dot(xb, w3_ref[hp])
    cos = cos_ref[...]
    sin = sin_ref[...]
    for i in range(2):
        qh = z[:, 192 * i:192 * i + DHEAD]
        kh = z[:, 192 * i + DHEAD:192 * i + 2 * DHEAD]
        vh = z[:, 192 * i + 2 * DHEAD:192 * i + 3 * DHEAD]
        qr = qh * cos + _pairswap(qh) * sin
        kr = kh * cos + _pairswap(kh) * sin
        q_ref[i] = qr.astype(bf16)
        k_ref[i] = kr
        kb_ref[i] = kr.astype(bf16)
        vb_ref[i] = vh.astype(bf16)


def _cmlp_kernel(k2_ref, v2_ref, kp_ref, vp_ref,
                 kfc_ref, kpj_ref, vfc_ref, vpj_ref, ck_ref, cv_ref):
    kin = k2_ref[0] + kp_ref[0]
    vin = v2_ref[0] + vp_ref[0]
    hk = jnp.square(jax.nn.relu(_dot(kin, kfc_ref[...])))
    ck_ref[0] = _dot(hk, kpj_ref[...]).astype(bf16)
    hv = jnp.square(jax.nn.relu(_dot(vin, vfc_ref[...])))
    cv_ref[0] = _dot(hv, vpj_ref[...]).astype(bf16)


def _cdef_kernel(q_ref, kw_ref, vw_ref, ck_ref, cv_ref, cmask_ref, emat_ref,
                 x_ref, wg_ref, bg_ref, cw_ref, cd_ref, ca_ref, w0_ref,
                 w1_ref, out_ref,
                 imp_scr, sel_scr, cout_scr, fout_scr, sout_scr):
    qt = pl.program_id(0)
    ph = pl.program_id(1)
    h = pl.program_id(2)
    qb = q_ref[h, pl.ds(qt * TQ, TQ), :]

    @pl.when(ph == 0)
    def _():
        sim = _dot_nt(qb, ck_ref[h]) * SCALE
        sim = jnp.where(cmask_ref[...] != 0, sim, NEG)
        m = jnp.max(sim, axis=1, keepdims=True)
        e = jnp.exp(sim - m)
        attn = e / jnp.sum(e, axis=1, keepdims=True)
        cout_scr[h] = _dot(attn, cv_ref[h])

        @pl.when(h == 0)
        def _():
            imp_scr[...] = jnp.zeros_like(imp_scr)
        imp_scr[...] += attn[:, :NBLK] * (1.0 / HEADS)

        @pl.when(h == HEADS - 1)
        def _():
            val = imp_scr[...]
            iota = jax.lax.broadcasted_iota(jnp.int32, (TQ, NBLK), 1)
            sel = jnp.zeros((TQ, NBLK), f32)
            for _ in range(NSEL):
                mx = jnp.max(val, axis=1, keepdims=True)
                cand = jnp.where(val == mx, iota, NBLK * 4)
                idx = jnp.min(cand, axis=1, keepdims=True)
                oh = iota == idx
                sel = sel + oh.astype(f32)
                val = jnp.where(oh, -1.0, val)
            sel_scr[...] = (_dot(sel.astype(bf16), emat_ref[...])
                            > 0.5).astype(jnp.int8)

    @pl.when(ph == 1)
    def _():
        def body(kt, carry):
            mf, lf, af = carry
            kb = kw_ref[h, pl.ds(kt * TK + WPAD, TK), :]
            vb = vw_ref[h, pl.ds(kt * TK + WPAD, TK), :]
            sim = _dot_nt(qb, kb) * SCALE
            fs = jnp.where(sel_scr[:, pl.ds(kt * TK, TK)] != 0, sim, NEG)
            mf2 = jnp.maximum(mf, jnp.max(fs, axis=1, keepdims=True))
            a = jnp.exp(mf - mf2)
            p = jnp.exp(fs - mf2)
            lf = lf * a + jnp.sum(p, axis=1, keepdims=True)
            af = af * a + _dot(p, vb)
            return mf2, lf, af

        init = (jnp.full((TQ, 1), NEG, f32), jnp.zeros((TQ, 1), f32),
                jnp.zeros((TQ, DHEAD), f32))
        mf, lf, af = jax.lax.fori_loop(0, qt, body, init)

        kb = kw_ref[h, pl.ds(qt * TK + WPAD, TK), :]
        vb = vw_ref[h, pl.ds(qt * TK + WPAD, TK), :]
        sim = _dot_nt(qb, kb) * SCALE
        dmask = (cd_ref[...] != 0) | ((ca_ref[...] != 0) &
                                      (sel_scr[:, pl.ds(qt * TK, TK)] != 0))
        fs = jnp.where(dmask, sim, NEG)
        mf2 = jnp.maximum(mf, jnp.max(fs, axis=1, keepdims=True))
        a = jnp.exp(mf - mf2)
        p = jnp.exp(fs - mf2)
        lf = lf * a + jnp.sum(p, axis=1, keepdims=True)
        af = af * a + _dot(p, vb)
        fout_scr[h] = af / lf

        kb = kw_ref[h, pl.ds(qt * TK, WCOLS), :]
        vb = vw_ref[h, pl.ds(qt * TK, WCOLS), :]
        ws = _dot_nt(qb, kb) * SCALE
        wm = jnp.where(qt == 0, w0_ref[...], w1_ref[...])
        ws = jnp.where(wm != 0, ws, NEG)
        mw = jnp.max(ws, axis=1, keepdims=True)
        pw = jnp.exp(ws - mw)
        sout_scr[h] = _dot(pw, vb) / jnp.sum(pw, axis=1, keepdims=True)

        @pl.when(h == HEADS - 1)
        def _():
            xb = x_ref[...]
            g = jax.nn.sigmoid(_dot(xb, wg_ref[...]) + bg_ref[...])
            acc = jnp.zeros((TQ, DIM), f32)
            for hh in range(HEADS):
                gc = g[:, 3 * hh:3 * hh + 1]
                gf = g[:, 3 * hh + 1:3 * hh + 2]
                gs = g[:, 3 * hh + 2:3 * hh + 3]
                y = (gc * cout_scr[hh] + gf * fout_scr[hh] +
                     gs * sout_scr[hh])
                acc = acc + _dot(y, cw_ref[hh])
            out_ref[...] = acc


def _build_tables():
    i8 = jnp.int8
    inv = 1.0 / (10000.0 ** (jnp.arange(0, DHEAD, 2, dtype=f32) / DHEAD))
    freqs = jnp.arange(T, dtype=f32)[:, None] * inv[None, :]
    c = jnp.cos(freqs)
    si = jnp.sin(freqs)
    cos = jnp.stack([c, c], axis=-1).reshape(T, DHEAD)
    sin = jnp.stack([-si, si], axis=-1).reshape(T, DHEAD)
    emat = (jnp.arange(NBLK)[:, None] ==
            (jnp.arange(T)[None, :] // CBS)).astype(bf16)
    tq = jnp.arange(T)[:, None]
    sc = jnp.arange(CKP)[None, :]
    cmask = (((sc < NBLK) & (CBS * sc + (CBS - 1) < tq)) |
             (sc == NBLK)).astype(i8)
    r = jnp.arange(TQ)[:, None]
    cc = jnp.arange(TK)[None, :]
    ca = (cc <= r)
    cd = (ca & ((r // SBS) == (cc // SBS))).astype(i8)
    ca = ca.astype(i8)
    cw = jnp.arange(WCOLS)[None, :]
    dt = r + WPAD - cw
    wbase = (dt >= 0) & (dt < WINDOW)
    w1 = wbase.astype(i8)
    w0 = (wbase & (cw >= WPAD)).astype(i8)
    return cos, sin, emat, cmask, cd, ca, w0, w1


def kernel(x, qkv_w, k_fc_w, k_proj_w, v_fc_w, v_proj_w, compress_mem_kv,
           k_pos, v_pos, strat_w, strat_b, combine_w):
    x2 = x.reshape(T, DIM)
    cos, sin, emat, cmask, cd, ca, w0, w1 = _build_tables()

    w3 = jnp.transpose(qkv_w.reshape(3, HEADS, DHEAD, DIM), (1, 3, 0, 2))
    w3 = w3.reshape(HEADS // 2, 2, DIM, 3 * DHEAD)
    w3 = w3.transpose(0, 2, 1, 3).reshape(HEADS // 2, DIM, 6 * DHEAD)
    q, k, kb16, vb16 = pl.pallas_call(
        _qkv_kernel,
        grid=(QT, HEADS // 2),
        in_specs=[
            pl.BlockSpec((TQ, DIM), lambda qt, h: (qt, 0)),
            pl.BlockSpec((HEADS // 2, DIM, 6 * DHEAD),
                         lambda qt, h: (0, 0, 0)),
            pl.BlockSpec((TQ, DHEAD), lambda qt, h: (qt, 0)),
            pl.BlockSpec((TQ, DHEAD), lambda qt, h: (qt, 0)),
        ],
        out_specs=[
            pl.BlockSpec((2, TQ, DHEAD), lambda qt, h: (h, qt, 0)),
            pl.BlockSpec((2, TQ, DHEAD), lambda qt, h: (h, qt, 0)),
            pl.BlockSpec((2, TQ, DHEAD), lambda qt, h: (h, qt, 0)),
            pl.BlockSpec((2, TQ, DHEAD), lambda qt, h: (h, qt, 0)),
        ],
        out_shape=[
            jax.ShapeDtypeStruct((HEADS, T, DHEAD), bf16),
            jax.ShapeDtypeStruct((HEADS, T, DHEAD), f32),
            jax.ShapeDtypeStruct((HEADS, T, DHEAD), bf16),
            jax.ShapeDtypeStruct((HEADS, T, DHEAD), bf16),
        ],
    )(x2, w3, cos, sin)

    k2 = k.reshape(HEADS, NBLK, CDIM)
    v2 = vb16.reshape(HEADS, NBLK, CDIM)
    kp = k_pos.reshape(HEADS, 1, CDIM)
    vp = v_pos.reshape(HEADS, 1, CDIM)
    TB = 128
    ck, cv = pl.pallas_call(
        _cmlp_kernel,
        grid=(HEADS, NBLK // TB),
        in_specs=[
            pl.BlockSpec((1, TB, CDIM), lambda h, b: (h, b, 0)),
            pl.BlockSpec((1, TB, CDIM), lambda h, b: (h, b, 0)),
            pl.BlockSpec((1, 1, CDIM), lambda h, b: (h, 0, 0)),
            pl.BlockSpec((1, 1, CDIM), lambda h, b: (h, 0, 0)),
            pl.BlockSpec((CDIM, HID), lambda h, b: (0, 0)),
            pl.BlockSpec((HID, DHEAD), lambda h, b: (0, 0)),
            pl.BlockSpec((CDIM, HID), lambda h, b: (0, 0)),
            pl.BlockSpec((HID, DHEAD), lambda h, b: (0, 0)),
        ],
        out_specs=[
            pl.BlockSpec((1, TB, DHEAD), lambda h, b: (h, b, 0)),
            pl.BlockSpec((1, TB, DHEAD), lambda h, b: (h, b, 0)),
        ],
        out_shape=[jax.ShapeDtypeStruct((HEADS, NBLK, DHEAD), bf16)] * 2,
    )(k2, v2, kp, vp, k_fc_w.T, k_proj_w.T, v_fc_w.T, v_proj_w.T)

    mem_k = compress_mem_kv[0].astype(bf16)
    mem_v = compress_mem_kv[1].astype(bf16)
    zpad = jnp.zeros((HEADS, CKP - NBLK - NMEM, DHEAD), bf16)
    ck_full = jnp.concatenate([ck, mem_k, zpad], axis=1)
    cv_full = jnp.concatenate([cv, mem_v, zpad], axis=1)
    kw = jnp.concatenate(
        [jnp.zeros((HEADS, WPAD, DHEAD), bf16), kb16,
         jnp.zeros((HEADS, TW - WPAD - T, DHEAD), bf16)], axis=1)
    vw = jnp.concatenate(
        [jnp.zeros((HEADS, WPAD, DHEAD), bf16), vb16,
         jnp.zeros((HEADS, TW - WPAD - T, DHEAD), bf16)], axis=1)
    wg = jnp.concatenate([strat_w.T, jnp.zeros((DIM, 128 - 3 * HEADS), f32)],
                         axis=1)
    bg = jnp.concatenate([strat_b, jnp.zeros((128 - 3 * HEADS,), f32)])
    bg = bg.reshape(1, 128)
    cw = combine_w.T.reshape(HEADS, DHEAD, DIM)

    out = pl.pallas_call(
        _cdef_kernel,
        grid=(QT, 2, HEADS),
        in_specs=[
            pl.BlockSpec((HEADS, T, DHEAD), lambda qt, ph, h: (0, 0, 0)),
            pl.BlockSpec((HEADS, TW, DHEAD), lambda qt, ph, h: (0, 0, 0)),
            pl.BlockSpec((HEADS, TW, DHEAD), lambda qt, ph, h: (0, 0, 0)),
            pl.BlockSpec((HEADS, CKP, DHEAD), lambda qt, ph, h: (0, 0, 0)),
            pl.BlockSpec((HEADS, CKP, DHEAD), lambda qt, ph, h: (0, 0, 0)),
            pl.BlockSpec((TQ, CKP), lambda qt, ph, h: (qt, 0)),
            pl.BlockSpec((NBLK, T), lambda qt, ph, h: (0, 0)),
            pl.BlockSpec((TQ, DIM), lambda qt, ph, h: (qt, 0)),
            pl.BlockSpec((DIM, 128), lambda qt, ph, h: (0, 0)),
            pl.BlockSpec((1, 128), lambda qt, ph, h: (0, 0)),
            pl.BlockSpec((HEADS, DHEAD, DIM), lambda qt, ph, h: (0, 0, 0)),
            pl.BlockSpec((TQ, TK), lambda qt, ph, h: (0, 0)),
            pl.BlockSpec((TQ, TK), lambda qt, ph, h: (0, 0)),
            pl.BlockSpec((TQ, WCOLS), lambda qt, ph, h: (0, 0)),
            pl.BlockSpec((TQ, WCOLS), lambda qt, ph, h: (0, 0)),
        ],
        out_specs=pl.BlockSpec((TQ, DIM), lambda qt, ph, h: (qt, 0)),
        out_shape=jax.ShapeDtypeStruct((T, DIM), f32),
        scratch_shapes=[
            pltpu.VMEM((TQ, NBLK), f32),
            pltpu.VMEM((TQ, T), jnp.int8),
            pltpu.VMEM((HEADS, TQ, DHEAD), f32),
            pltpu.VMEM((HEADS, TQ, DHEAD), f32),
            pltpu.VMEM((HEADS, TQ, DHEAD), f32),
        ],
    )(q, kw, vw, ck_full, cv_full, cmask, emat, x2, wg, bg, cw,
      cd, ca, w0, w1)

    return out.reshape(B, T, DIM)


# speedup vs baseline: 1.3652x; 1.3652x over previous
import jax
import jax.numpy as jnp
from jax.experimental import pallas as pl
from jax.experimental.pallas import tpu as pltpu

B, T, DIM = 1, 2048, 768
HEADS, DHEAD = 12, 64
HDIM = HEADS * DHEAD
CBS, SBS = 4, 4
NSEL, NMEM = 4, 1
WINDOW = 32
SCALE = 0.12
CDIM = CBS * DHEAD
HID = CDIM * 4
NBLK = T // CBS

TQ = 512
TK = 512
QT = T // TQ
CKP = 640
NEG = -1e30
WPAD = 32
WCOLS = 576
TW = WPAD + T + (WCOLS - TK - WPAD)

f32 = jnp.float32
bf16 = jnp.bfloat16


def _dot(a, b, precision=jax.lax.Precision.DEFAULT):
    return jax.lax.dot_general(a, b, (((1,), (0,)), ((), ())),
                               preferred_element_type=f32,
                               precision=precision)


def _dot_nt(a, b, precision=jax.lax.Precision.DEFAULT):
    return jax.lax.dot_general(a, b, (((1,), (1,)), ((), ())),
                               preferred_element_type=f32,
                               precision=precision)


def _pairswap(x):
    even = jax.lax.broadcasted_iota(jnp.int32, x.shape, 1) % 2 == 0
    return jnp.where(even, jnp.roll(x, -1, axis=1), jnp.roll(x, 1, axis=1))


def _qkv_kernel(x_ref, w3_ref, cos_ref, sin_ref,
                q_ref, k_ref, kb_ref, vb_ref):
    hp = pl.program_id(1)
    xb = x_ref[...]
    z = _dot(xb, w3_ref[hp])
    cos = cos_ref[...]
    sin = sin_ref[...]
    for i in range(2):
        qh = z[:, 192 * i:192 * i + DHEAD]
        kh = z[:, 192 * i + DHEAD:192 * i + 2 * DHEAD]
        vh = z[:, 192 * i + 2 * DHEAD:192 * i + 3 * DHEAD]
        qr = qh * cos + _pairswap(qh) * sin
        kr = kh * cos + _pairswap(kh) * sin
        q_ref[i] = qr.astype(bf16)
        k_ref[i] = kr
        kb_ref[i] = kr.astype(bf16)
        vb_ref[i] = vh.astype(bf16)


def _cmlp_kernel(k2_ref, v2_ref, kp_ref, vp_ref,
                 kfc_ref, kpj_ref, vfc_ref, vpj_ref, ck_ref, cv_ref):
    kin = k2_ref[0] + kp_ref[0]
    vin = v2_ref[0] + vp_ref[0]
    hk = jnp.square(jax.nn.relu(_dot(kin, kfc_ref[...])))
    ck_ref[0] = _dot(hk, kpj_ref[...]).astype(bf16)
    hv = jnp.square(jax.nn.relu(_dot(vin, vfc_ref[...])))
    cv_ref[0] = _dot(hv, vpj_ref[...]).astype(bf16)


def _cdef_kernel(q_ref, kw_ref, vw_ref, ck_ref, cv_ref, cmask_ref, emat_ref,
                 x_ref, wg_ref, bg_ref, cw_ref, cd_ref, ca_ref, w0_ref,
                 w1_ref, out_ref,
                 imp_scr, sel_scr, cout_scr, fout_scr, sout_scr):
    qt = pl.program_id(0)
    ph = pl.program_id(1)
    h = pl.program_id(2)
    qb = q_ref[h, pl.ds(qt * TQ, TQ), :]

    @pl.when(ph == 0)
    def _():
        sim = _dot_nt(qb, ck_ref[h]) * SCALE
        sim = jnp.where(cmask_ref[...] != 0, sim, NEG)
        m = jnp.max(sim, axis=1, keepdims=True)
        e = jnp.exp(sim - m)
        attn = e / jnp.sum(e, axis=1, keepdims=True)
        cout_scr[h] = _dot(attn, cv_ref[h])

        @pl.when(h == 0)
        def _():
            imp_scr[...] = jnp.zeros_like(imp_scr)
        imp_scr[...] += attn[:, :NBLK] * (1.0 / HEADS)

        @pl.when(h == HEADS - 1)
        def _():
            val = imp_scr[...]
            iota = jax.lax.broadcasted_iota(jnp.int32, (TQ, NBLK), 1)
            sel = jnp.zeros((TQ, NBLK), f32)
            for _ in range(NSEL):
                mx = jnp.max(val, axis=1, keepdims=True)
                cand = jnp.where(val == mx, iota, NBLK * 4)
                idx = jnp.min(cand, axis=1, keepdims=True)
                oh = iota == idx
                sel = sel + oh.astype(f32)
                val = jnp.where(oh, -1.0, val)
            sel_scr[...] = (_dot(sel.astype(bf16), emat_ref[...])
                            > 0.5).astype(jnp.int8)

    @pl.when(ph == 1)
    def _():
        def body(kt, carry):
            mf, lf, af = carry
            kb = kw_ref[h, pl.ds(kt * TK + WPAD, TK), :]
            vb = vw_ref[h, pl.ds(kt * TK + WPAD, TK), :]
            sim = _dot_nt(qb, kb) * SCALE
            fs = jnp.where(sel_scr[:, pl.ds(kt * TK, TK)] != 0, sim, NEG)
            mf2 = jnp.maximum(mf, jnp.max(fs, axis=1, keepdims=True))
            a = jnp.exp(mf - mf2)
            p = jnp.exp(fs - mf2)
            lf = lf * a + jnp.sum(p, axis=1, keepdims=True)
            af = af * a + _dot(p, vb)
            return mf2, lf, af

        init = (jnp.full((TQ, 1), NEG, f32), jnp.zeros((TQ, 1), f32),
                jnp.zeros((TQ, DHEAD), f32))
        mf, lf, af = jax.lax.fori_loop(0, qt, body, init)

        kb = kw_ref[h, pl.ds(qt * TK + WPAD, TK), :]
        vb = vw_ref[h, pl.ds(qt * TK + WPAD, TK), :]
        sim = _dot_nt(qb, kb) * SCALE
        dmask = (cd_ref[...] != 0) | ((ca_ref[...] != 0) &
                                      (sel_scr[:, pl.ds(qt * TK, TK)] != 0))
        fs = jnp.where(dmask, sim, NEG)
        mf2 = jnp.maximum(mf, jnp.max(fs, axis=1, keepdims=True))
        a = jnp.exp(mf - mf2)
        p = jnp.exp(fs - mf2)
        lf = lf * a + jnp.sum(p, axis=1, keepdims=True)
        af = af * a + _dot(p, vb)
        fout_scr[h] = af / lf

        kb = kw_ref[h, pl.ds(qt * TK, WCOLS), :]
        vb = vw_ref[h, pl.ds(qt * TK, WCOLS), :]
        ws = _dot_nt(qb, kb) * SCALE
        wm = jnp.where(qt == 0, w0_ref[...], w1_ref[...])
        ws = jnp.where(wm != 0, ws, NEG)
        mw = jnp.max(ws, axis=1, keepdims=True)
        pw = jnp.exp(ws - mw)
        sout_scr[h] = _dot(pw, vb) / jnp.sum(pw, axis=1, keepdims=True)

        @pl.when(h == HEADS - 1)
        def _():
            xb = x_ref[...]
            g = jax.nn.sigmoid(_dot(xb, wg_ref[...]) + bg_ref[...])
            acc = jnp.zeros((TQ, DIM), f32)
            for hh in range(HEADS):
                gc = g[:, 3 * hh:3 * hh + 1]
                gf = g[:, 3 * hh + 1:3 * hh + 2]
                gs = g[:, 3 * hh + 2:3 * hh + 3]
                y = (gc * cout_scr[hh] + gf * fout_scr[hh] +
                     gs * sout_scr[hh])
                acc = acc + _dot(y, cw_ref[hh])
            out_ref[...] = acc


def _build_tables():
    i8 = jnp.int8
    inv = 1.0 / (10000.0 ** (jnp.arange(0, DHEAD, 2, dtype=f32) / DHEAD))
    freqs = jnp.arange(T, dtype=f32)[:, None] * inv[None, :]
    c = jnp.cos(freqs)
    si = jnp.sin(freqs)
    cos = jnp.stack([c, c], axis=-1).reshape(T, DHEAD)
    sin = jnp.stack([-si, si], axis=-1).reshape(T, DHEAD)
    emat = (jnp.arange(NBLK)[:, None] ==
            (jnp.arange(T)[None, :] // CBS)).astype(bf16)
    tq = jnp.arange(T)[:, None]
    sc = jnp.arange(CKP)[None, :]
    cmask = (((sc < NBLK) & (CBS * sc + (CBS - 1) < tq)) |
             (sc == NBLK)).astype(i8)
    r = jnp.arange(TQ)[:, None]
    cc = jnp.arange(TK)[None, :]
    ca = (cc <= r)
    cd = (ca & ((r // SBS) == (cc // SBS))).astype(i8)
    ca = ca.astype(i8)
    cw = jnp.arange(WCOLS)[None, :]
    dt = r + WPAD - cw
    wbase = (dt >= 0) & (dt < WINDOW)
    w1 = wbase.astype(i8)
    w0 = (wbase & (cw >= WPAD)).astype(i8)
    return cos, sin, emat, cmask, cd, ca, w0, w1


def kernel(x, qkv_w, k_fc_w, k_proj_w, v_fc_w, v_proj_w, compress_mem_kv,
           k_pos, v_pos, strat_w, strat_b, combine_w):
    x2 = x.reshape(T, DIM)
    cos, sin, emat, cmask, cd, ca, w0, w1 = _build_tables()

    w3 = jnp.transpose(qkv_w.reshape(3, HEADS, DHEAD, DIM), (1, 3, 0, 2))
    w3 = w3.reshape(HEADS // 2, 2, DIM, 3 * DHEAD)
    w3 = w3.transpose(0, 2, 1, 3).reshape(HEADS // 2, DIM, 6 * DHEAD)
    q, k, kb16, vb16 = pl.pallas_call(
        _qkv_kernel,
        grid=(QT, HEADS // 2),
        in_specs=[
            pl.BlockSpec((TQ, DIM), lambda qt, h: (qt, 0)),
            pl.BlockSpec((HEADS // 2, DIM, 6 * DHEAD),
                         lambda qt, h: (0, 0, 0)),
            pl.BlockSpec((TQ, DHEAD), lambda qt, h: (qt, 0)),
            pl.BlockSpec((TQ, DHEAD), lambda qt, h: (qt, 0)),
        ],
        out_specs=[
            pl.BlockSpec((2, TQ, DHEAD), lambda qt, h: (h, qt, 0)),
            pl.BlockSpec((2, TQ, DHEAD), lambda qt, h: (h, qt, 0)),
            pl.BlockSpec((2, TQ, DHEAD), lambda qt, h: (h, qt, 0)),
            pl.BlockSpec((2, TQ, DHEAD), lambda qt, h: (h, qt, 0)),
        ],
        out_shape=[
            jax.ShapeDtypeStruct((HEADS, T, DHEAD), bf16),
            jax.ShapeDtypeStruct((HEADS, T, DHEAD), f32),
            jax.ShapeDtypeStruct((HEADS, T, DHEAD), bf16),
            jax.ShapeDtypeStruct((HEADS, T, DHEAD), bf16),
        ],
    )(x2, w3, cos, sin)

    k2 = k.reshape(HEADS, NBLK, CDIM)
    v2 = vb16.reshape(HEADS, NBLK, CDIM)
    kp = k_pos.reshape(HEADS, 1, CDIM)
    vp = v_pos.reshape(HEADS, 1, CDIM)
    TB = 512
    ck, cv = pl.pallas_call(
        _cmlp_kernel,
        grid=(HEADS, NBLK // TB),
        in_specs=[
            pl.BlockSpec((1, TB, CDIM), lambda h, b: (h, b, 0)),
            pl.BlockSpec((1, TB, CDIM), lambda h, b: (h, b, 0)),
            pl.BlockSpec((1, 1, CDIM), lambda h, b: (h, 0, 0)),
            pl.BlockSpec((1, 1, CDIM), lambda h, b: (h, 0, 0)),
            pl.BlockSpec((CDIM, HID), lambda h, b: (0, 0)),
            pl.BlockSpec((HID, DHEAD), lambda h, b: (0, 0)),
            pl.BlockSpec((CDIM, HID), lambda h, b: (0, 0)),
            pl.BlockSpec((HID, DHEAD), lambda h, b: (0, 0)),
        ],
        out_specs=[
            pl.BlockSpec((1, TB, DHEAD), lambda h, b: (h, b, 0)),
            pl.BlockSpec((1, TB, DHEAD), lambda h, b: (h, b, 0)),
        ],
        out_shape=[jax.ShapeDtypeStruct((HEADS, NBLK, DHEAD), bf16)] * 2,
    )(k2, v2, kp, vp, k_fc_w.T, k_proj_w.T, v_fc_w.T, v_proj_w.T)

    mem_k = compress_mem_kv[0].astype(bf16)
    mem_v = compress_mem_kv[1].astype(bf16)
    zpad = jnp.zeros((HEADS, CKP - NBLK - NMEM, DHEAD), bf16)
    ck_full = jnp.concatenate([ck, mem_k, zpad], axis=1)
    cv_full = jnp.concatenate([cv, mem_v, zpad], axis=1)
    kw = jnp.concatenate(
        [jnp.zeros((HEADS, WPAD, DHEAD), bf16), kb16,
         jnp.zeros((HEADS, TW - WPAD - T, DHEAD), bf16)], axis=1)
    vw = jnp.concatenate(
        [jnp.zeros((HEADS, WPAD, DHEAD), bf16), vb16,
         jnp.zeros((HEADS, TW - WPAD - T, DHEAD), bf16)], axis=1)
    wg = jnp.concatenate([strat_w.T, jnp.zeros((DIM, 128 - 3 * HEADS), f32)],
                         axis=1)
    bg = jnp.concatenate([strat_b, jnp.zeros((128 - 3 * HEADS,), f32)])
    bg = bg.reshape(1, 128)
    cw = combine_w.T.reshape(HEADS, DHEAD, DIM)

    out = pl.pallas_call(
        _cdef_kernel,
        grid=(QT, 2, HEADS),
        in_specs=[
            pl.BlockSpec((HEADS, T, DHEAD), lambda qt, ph, h: (0, 0, 0)),
            pl.BlockSpec((HEADS, TW, DHEAD), lambda qt, ph, h: (0, 0, 0)),
            pl.BlockSpec((HEADS, TW, DHEAD), lambda qt, ph, h: (0, 0, 0)),
            pl.BlockSpec((HEADS, CKP, DHEAD), lambda qt, ph, h: (0, 0, 0)),
            pl.BlockSpec((HEADS, CKP, DHEAD), lambda qt, ph, h: (0, 0, 0)),
            pl.BlockSpec((TQ, CKP), lambda qt, ph, h: (qt, 0)),
            pl.BlockSpec((NBLK, T), lambda qt, ph, h: (0, 0)),
            pl.BlockSpec((TQ, DIM), lambda qt, ph, h: (qt, 0)),
            pl.BlockSpec((DIM, 128), lambda qt, ph, h: (0, 0)),
            pl.BlockSpec((1, 128), lambda qt, ph, h: (0, 0)),
            pl.BlockSpec((HEADS, DHEAD, DIM), lambda qt, ph, h: (0, 0, 0)),
            pl.BlockSpec((TQ, TK), lambda qt, ph, h: (0, 0)),
            pl.BlockSpec((TQ, TK), lambda qt, ph, h: (0, 0)),
            pl.BlockSpec((TQ, WCOLS), lambda qt, ph, h: (0, 0)),
            pl.BlockSpec((TQ, WCOLS), lambda qt, ph, h: (0, 0)),
        ],
        out_specs=pl.BlockSpec((TQ, DIM), lambda qt, ph, h: (qt, 0)),
        out_shape=jax.ShapeDtypeStruct((T, DIM), f32),
        scratch_shapes=[
            pltpu.VMEM((TQ, NBLK), f32),
            pltpu.VMEM((TQ, T), jnp.int8),
            pltpu.VMEM((HEADS, TQ, DHEAD), f32),
            pltpu.VMEM((HEADS, TQ, DHEAD), f32),
            pltpu.VMEM((HEADS, TQ, DHEAD), f32),
        ],
    )(q, kw, vw, ck_full, cv_full, cmask, emat, x2, wg, bg, cw,
      cd, ca, w0, w1)

    return out.reshape(B, T, DIM)
